# Initial kernel scaffold; baseline (speedup 1.0000x reference)
import jax
import jax.numpy as jnp
from jax.experimental import pallas as pl


def kernel(x, edge_index, W1_0, b1_0, W2_0, b2_0, gamma_0, beta_0, mean_0, var_0, W1_1, b1_1, W2_1, b2_1, gamma_1, beta_1, mean_1, var_1, W1_2, b1_2, W2_2, b2_2, gamma_2, beta_2, mean_2, var_2, Wc, bc):
    raise NotImplementedError("write your pallas kernel here")


# speedup vs baseline: 2.8627x; 2.8627x over previous
import functools

import jax
import jax.numpy as jnp
from jax import lax
from jax.experimental import pallas as pl
from jax.experimental.pallas import tpu as pltpu
from jax.experimental.pallas import tpu_sc as plsc

N = 10000
D = 128
E = 320000
D_OUT = 64

NC = 2
NS = 16
NW = NC * NS
C = 128
EPW = E // NW
CH = 80
EPW_PAD = CH * C
NPAD = 10240
TRASH = N
RPT = NPAD // NS
BLK = 1280

def _build_sc_segment_sum():
    mesh = plsc.VectorSubcoreMesh(
        core_axis_name="c", subcore_axis_name="s", num_cores=NC, num_subcores=NS
    )

    @functools.partial(
        pl.kernel,
        out_type=(
            jax.ShapeDtypeStruct((NPAD, D), jnp.float32),
            jax.ShapeDtypeStruct((NPAD, D), jnp.float32),
        ),
        mesh=mesh,
        scratch_types=[
            pltpu.VMEM((CH, C), jnp.int32),
            pltpu.VMEM((CH, C), jnp.int32),
            pltpu.VMEM((C, D), jnp.float32),
            pltpu.VMEM_SHARED((NPAD, D), jnp.float32),
            pltpu.SemaphoreType.DMA,
        ],
    )
    def sc_segment_sum(x_hbm, srci_hbm, dsti_hbm, zeros_hbm,
                       out0_hbm, out1_hbm,
                       srci_v, dsti_v, rows_v, acc, gsem):
        c = lax.axis_index("c")
        s = lax.axis_index("s")
        wid = c * NS + s
        rows = pl.ds(s * RPT, RPT)
        pltpu.sync_copy(srci_hbm.at[wid], srci_v)
        pltpu.sync_copy(dsti_hbm.at[wid], dsti_v)
        pltpu.sync_copy(zeros_hbm, acc.at[rows])
        plsc.subcore_barrier()

        @pl.loop(0, CH)
        def _(j):
            pltpu.async_copy(x_hbm.at[srci_v.at[j]], rows_v, gsem).wait()
            pltpu.sync_copy(rows_v, acc.at[dsti_v.at[j]], add=True)

        plsc.subcore_barrier()

        @pl.when(c == 0)
        def _():
            pltpu.sync_copy(acc.at[rows], out0_hbm.at[rows])

        @pl.when(c == 1)
        def _():
            pltpu.sync_copy(acc.at[rows], out1_hbm.at[rows])

    return sc_segment_sum


_sc_segment_sum_cache = []


def _sc_segment_sum(*args):
    if not _sc_segment_sum_cache:
        _sc_segment_sum_cache.append(_build_sc_segment_sum())
    return _sc_segment_sum_cache[0](*args)


def _mlp_body(x_ref, a0_ref, a1_ref, w1_ref, b1_ref, w2_ref, b2_ref,
              g_ref, bt_ref, mn_ref, vr_ref, o_ref):
    a = x_ref[...] + a0_ref[...] + a1_ref[...]
    t = lax.dot_general(a, w1_ref[...], (((1,), (0,)), ((), ())),
                        preferred_element_type=jnp.float32)
    t = jnp.maximum(t + b1_ref[...], 0.0)
    h = lax.dot_general(t, w2_ref[...], (((1,), (0,)), ((), ())),
                        preferred_element_type=jnp.float32)
    h = h + b2_ref[...]
    h = (h - mn_ref[...]) * lax.rsqrt(vr_ref[...] + 1e-5) * g_ref[...] + bt_ref[...]
    o_ref[...] = jnp.maximum(h, 0.0)


_vec_spec = pl.BlockSpec((1, D), lambda i: (0, 0))
_w_spec = pl.BlockSpec((D, D), lambda i: (0, 0))
_row_spec = pl.BlockSpec((BLK, D), lambda i: (i, 0))

_mlp = pl.pallas_call(
    _mlp_body,
    grid=(NPAD // BLK,),
    in_specs=[_row_spec, _row_spec, _row_spec,
              _w_spec, _vec_spec, _w_spec, _vec_spec,
              _vec_spec, _vec_spec, _vec_spec, _vec_spec],
    out_specs=_row_spec,
    out_shape=jax.ShapeDtypeStruct((NPAD, D), jnp.float32),
)


def _mlp_pool_body(x_ref, a0_ref, a1_ref, w1_ref, b1_ref, w2_ref, b2_ref,
                   g_ref, bt_ref, mn_ref, vr_ref, wc_ref, bc_ref, o_ref):
    i = pl.program_id(0)
    a = x_ref[...] + a0_ref[...] + a1_ref[...]
    t = lax.dot_general(a, w1_ref[...], (((1,), (0,)), ((), ())),
                        preferred_element_type=jnp.float32)
    t = jnp.maximum(t + b1_ref[...], 0.0)
    h = lax.dot_general(t, w2_ref[...], (((1,), (0,)), ((), ())),
                        preferred_element_type=jnp.float32)
    h = h + b2_ref[...]
    h = (h - mn_ref[...]) * lax.rsqrt(vr_ref[...] + 1e-5) * g_ref[...] + bt_ref[...]
    h = jnp.maximum(h, 0.0)
    rid = lax.broadcasted_iota(jnp.int32, (BLK, 1), 0) + i * BLK
    part = jnp.sum(jnp.where(rid < N, h, 0.0), axis=0, keepdims=True)

    @pl.when(i == 0)
    def _():
        o_ref[...] = part

    @pl.when(i > 0)
    def _():
        o_ref[...] = o_ref[...] + part

    @pl.when(i == NPAD // BLK - 1)
    def _():
        pooled = o_ref[...]
        o_ref[...] = lax.dot_general(
            pooled, wc_ref[...], (((1,), (0,)), ((), ())),
            preferred_element_type=jnp.float32) + bc_ref[...]


_mlp_pool = pl.pallas_call(
    _mlp_pool_body,
    grid=(NPAD // BLK,),
    in_specs=[_row_spec, _row_spec, _row_spec,
              _w_spec, _vec_spec, _w_spec, _vec_spec,
              _vec_spec, _vec_spec, _vec_spec, _vec_spec,
              _w_spec, _vec_spec],
    out_specs=pl.BlockSpec((1, D), lambda i: (0, 0)),
    out_shape=jax.ShapeDtypeStruct((1, D), jnp.float32),
)


def kernel(x, edge_index, W1_0, b1_0, W2_0, b2_0, gamma_0, beta_0, mean_0,
           var_0, W1_1, b1_1, W2_1, b2_1, gamma_1, beta_1, mean_1, var_1,
           W1_2, b1_2, W2_2, b2_2, gamma_2, beta_2, mean_2, var_2, Wc, bc):
    pad_e = NW * EPW_PAD - E
    src = jnp.concatenate([edge_index[0], jnp.zeros((pad_e,), jnp.int32)])
    dst = jnp.concatenate([edge_index[1], jnp.full((pad_e,), TRASH, jnp.int32)])
    srci = src.reshape(NW, CH, C)
    dsti = dst.reshape(NW, CH, C)
    zeros = jnp.zeros((RPT, D), jnp.float32)
    h = jnp.concatenate([x, jnp.zeros((NPAD - N, D), jnp.float32)], axis=0)

    params = [
        (W1_0, b1_0, W2_0, b2_0, gamma_0, beta_0, mean_0, var_0),
        (W1_1, b1_1, W2_1, b2_1, gamma_1, beta_1, mean_1, var_1),
        (W1_2, b1_2, W2_2, b2_2, gamma_2, beta_2, mean_2, var_2),
    ]

    for li, (W1, b1, W2, b2, g, bt, mn, vr) in enumerate(params):
        a0, a1 = _sc_segment_sum(h, srci, dsti, zeros)
        vecs = [v.reshape(1, D) for v in (b1, b2, g, bt, mn, vr)]
        if li < 2:
            h = _mlp(h, a0, a1, W1, vecs[0], W2, vecs[1],
                     vecs[2], vecs[3], vecs[4], vecs[5])
        else:
            wc_pad = jnp.concatenate(
                [Wc, jnp.zeros((D, D - D_OUT), jnp.float32)], axis=1)
            bc_pad = jnp.concatenate(
                [bc, jnp.zeros((D - D_OUT,), jnp.float32)]).reshape(1, D)
            out = _mlp_pool(h, a0, a1, W1, vecs[0], W2, vecs[1],
                            vecs[2], vecs[3], vecs[4], vecs[5],
                            wc_pad, bc_pad)
    return out[:, :D_OUT]


# speedup vs baseline: 3.8558x; 1.3469x over previous
import functools

import jax
import jax.numpy as jnp
from jax import lax
from jax.experimental import pallas as pl
from jax.experimental.pallas import tpu as pltpu
from jax.experimental.pallas import tpu_sc as plsc

N = 10000
D = 128
E = 320000
D_OUT = 64

NC = 2
NS = 16
NW = NC * NS
C = 125
CH = 80
CL = 128
RPT = 632
RPT_LAST = N - (NS - 1) * RPT
BLK = 2000


def _build_sc_segment_sum():
    mesh = plsc.VectorSubcoreMesh(
        core_axis_name="c", subcore_axis_name="s", num_cores=NC, num_subcores=NS
    )

    @functools.partial(
        pl.kernel,
        out_type=(
            jax.ShapeDtypeStruct((N, D), jnp.float32),
            jax.ShapeDtypeStruct((N, D), jnp.float32),
        ),
        mesh=mesh,
        scratch_types=[
            pltpu.VMEM((4, CL), jnp.int32),
            pltpu.VMEM((CH, C), jnp.int32),
            pltpu.VMEM((2, CL, D), jnp.float32),
            pltpu.VMEM_SHARED((N, D), jnp.float32),
            pltpu.SemaphoreType.DMA,
            pltpu.SemaphoreType.DMA,
            pltpu.SemaphoreType.DMA,
            pltpu.SemaphoreType.DMA,
            pltpu.SemaphoreType.DMA,
            pltpu.SemaphoreType.DMA,
        ],
    )
    def sc_segment_sum(x_hbm, srci_hbm, dsti_hbm, zeros_hbm,
                       out0_hbm, out1_hbm,
                       iring, dsti_v, rows_v, acc,
                       isem0, isem1, isem2, isem3, gsem0, gsem1):
        c = lax.axis_index("c")
        s = lax.axis_index("s")
        wid = c * NS + s
        isems = (isem0, isem1, isem2, isem3)
        gsems = (gsem0, gsem1)

        def idx_fetch(jj, slot):
            return pltpu.make_async_copy(
                srci_hbm.at[wid, jj], iring.at[slot], isems[slot])

        def gather(jj_slot, b):
            return pltpu.make_async_copy(
                x_hbm.at[iring.at[jj_slot]], rows_v.at[b], gsems[b])

        idx_fetch(0, 0).start()
        idx_fetch(1, 1).start()
        pltpu.sync_copy(dsti_hbm.at[wid], dsti_v)
        idx_fetch(0, 0).wait()
        gather(0, 0).start()

        @pl.when(s < NS - 1)
        def _():
            pltpu.sync_copy(zeros_hbm, acc.at[pl.ds(s * RPT, RPT)])

        @pl.when(s == NS - 1)
        def _():
            pltpu.sync_copy(zeros_hbm.at[pl.ds(0, RPT_LAST)],
                            acc.at[pl.ds((NS - 1) * RPT, RPT_LAST)])

        plsc.subcore_barrier()

        @pl.loop(0, CH, step=4, unroll=1)
        def _(j):
            for q in range(4):
                jj = j + q
                b = q % 2
                gather(q, b).wait()

                @pl.when(jj + 2 < CH)
                def _():
                    idx_fetch(jj + 2, (q + 2) % 4).start()

                @pl.when(jj + 1 < CH)
                def _():
                    idx_fetch(jj + 1, (q + 1) % 4).wait()
                    gather((q + 1) % 4, 1 - b).start()

                pltpu.sync_copy(rows_v.at[b].at[pl.ds(0, C)],
                                acc.at[dsti_v.at[jj]], add=True)

        plsc.subcore_barrier()

        out_hbms = (out0_hbm, out1_hbm)
        for ci in range(NC):

            @pl.when(c == ci)
            def _(ci=ci):

                @pl.when(s < NS - 1)
                def _():
                    rows = pl.ds(s * RPT, RPT)
                    pltpu.sync_copy(acc.at[rows], out_hbms[ci].at[rows])

                @pl.when(s == NS - 1)
                def _():
                    rows = pl.ds((NS - 1) * RPT, RPT_LAST)
                    pltpu.sync_copy(acc.at[rows], out_hbms[ci].at[rows])

    return sc_segment_sum


_sc_segment_sum_cache = []


def _sc_segment_sum(*args):
    if not _sc_segment_sum_cache:
        _sc_segment_sum_cache.append(_build_sc_segment_sum())
    return _sc_segment_sum_cache[0](*args)


def _mlp_body(x_ref, a0_ref, a1_ref, w1_ref, b1_ref, w2_ref, b2_ref,
              g_ref, bt_ref, mn_ref, vr_ref, o_ref):
    a = x_ref[...] + a0_ref[...] + a1_ref[...]
    t = lax.dot_general(a, w1_ref[...], (((1,), (0,)), ((), ())),
                        preferred_element_type=jnp.float32)
    t = jnp.maximum(t + b1_ref[...], 0.0)
    h = lax.dot_general(t, w2_ref[...], (((1,), (0,)), ((), ())),
                        preferred_element_type=jnp.float32)
    h = h + b2_ref[...]
    h = (h - mn_ref[...]) * lax.rsqrt(vr_ref[...] + 1e-5) * g_ref[...] + bt_ref[...]
    o_ref[...] = jnp.maximum(h, 0.0)


_vec_spec = pl.BlockSpec((1, D), lambda i: (0, 0))
_w_spec = pl.BlockSpec((D, D), lambda i: (0, 0))
_row_spec = pl.BlockSpec((BLK, D), lambda i: (i, 0))

_mlp = pl.pallas_call(
    _mlp_body,
    grid=(N // BLK,),
    in_specs=[_row_spec, _row_spec, _row_spec,
              _w_spec, _vec_spec, _w_spec, _vec_spec,
              _vec_spec, _vec_spec, _vec_spec, _vec_spec],
    out_specs=_row_spec,
    out_shape=jax.ShapeDtypeStruct((N, D), jnp.float32),
)


def _mlp_pool_body(x_ref, a0_ref, a1_ref, w1_ref, b1_ref, w2_ref, b2_ref,
                   g_ref, bt_ref, mn_ref, vr_ref, wc_ref, bc_ref, o_ref):
    i = pl.program_id(0)
    a = x_ref[...] + a0_ref[...] + a1_ref[...]
    t = lax.dot_general(a, w1_ref[...], (((1,), (0,)), ((), ())),
                        preferred_element_type=jnp.float32)
    t = jnp.maximum(t + b1_ref[...], 0.0)
    h = lax.dot_general(t, w2_ref[...], (((1,), (0,)), ((), ())),
                        preferred_element_type=jnp.float32)
    h = h + b2_ref[...]
    h = (h - mn_ref[...]) * lax.rsqrt(vr_ref[...] + 1e-5) * g_ref[...] + bt_ref[...]
    h = jnp.maximum(h, 0.0)
    part = jnp.sum(h, axis=0, keepdims=True)

    @pl.when(i == 0)
    def _():
        o_ref[...] = part

    @pl.when(i > 0)
    def _():
        o_ref[...] = o_ref[...] + part

    @pl.when(i == N // BLK - 1)
    def _():
        pooled = o_ref[...]
        o_ref[...] = lax.dot_general(
            pooled, wc_ref[...], (((1,), (0,)), ((), ())),
            preferred_element_type=jnp.float32) + bc_ref[...]


_mlp_pool = pl.pallas_call(
    _mlp_pool_body,
    grid=(N // BLK,),
    in_specs=[_row_spec, _row_spec, _row_spec,
              _w_spec, _vec_spec, _w_spec, _vec_spec,
              _vec_spec, _vec_spec, _vec_spec, _vec_spec,
              _w_spec, _vec_spec],
    out_specs=pl.BlockSpec((1, D), lambda i: (0, 0)),
    out_shape=jax.ShapeDtypeStruct((1, D), jnp.float32),
)


def kernel(x, edge_index, W1_0, b1_0, W2_0, b2_0, gamma_0, beta_0, mean_0,
           var_0, W1_1, b1_1, W2_1, b2_1, gamma_1, beta_1, mean_1, var_1,
           W1_2, b1_2, W2_2, b2_2, gamma_2, beta_2, mean_2, var_2, Wc, bc):
    srci = jnp.concatenate(
        [edge_index[0].reshape(NW, CH, C),
         jnp.zeros((NW, CH, CL - C), jnp.int32)], axis=2)
    dsti = edge_index[1].reshape(NW, CH, C)
    zeros = jnp.zeros((RPT, D), jnp.float32)
    h = x

    params = [
        (W1_0, b1_0, W2_0, b2_0, gamma_0, beta_0, mean_0, var_0),
        (W1_1, b1_1, W2_1, b2_1, gamma_1, beta_1, mean_1, var_1),
        (W1_2, b1_2, W2_2, b2_2, gamma_2, beta_2, mean_2, var_2),
    ]

    for li, (W1, b1, W2, b2, g, bt, mn, vr) in enumerate(params):
        a0, a1 = _sc_segment_sum(h, srci, dsti, zeros)
        vecs = [v.reshape(1, D) for v in (b1, b2, g, bt, mn, vr)]
        if li < 2:
            h = _mlp(h, a0, a1, W1, vecs[0], W2, vecs[1],
                     vecs[2], vecs[3], vecs[4], vecs[5])
        else:
            wc_pad = jnp.concatenate(
                [Wc, jnp.zeros((D, D - D_OUT), jnp.float32)], axis=1)
            bc_pad = jnp.concatenate(
                [bc, jnp.zeros((D - D_OUT,), jnp.float32)]).reshape(1, D)
            out = _mlp_pool(h, a0, a1, W1, vecs[0], W2, vecs[1],
                            vecs[2], vecs[3], vecs[4], vecs[5],
                            wc_pad, bc_pad)
    return out[:, :D_OUT]


# speedup vs baseline: 3.8586x; 1.0007x over previous
import functools

import jax
import jax.numpy as jnp
from jax import lax
from jax.experimental import pallas as pl
from jax.experimental.pallas import tpu as pltpu
from jax.experimental.pallas import tpu_sc as plsc

N = 10000
D = 128
E = 320000
D_OUT = 64

NC = 2
NS = 16
NW = NC * NS
C = 125
CH = 80
CL = 128
RPT = 632
RPT_LAST = N - (NS - 1) * RPT
BLK = 2000


def _build_sc_segment_sum():
    mesh = plsc.VectorSubcoreMesh(
        core_axis_name="c", subcore_axis_name="s", num_cores=NC, num_subcores=NS
    )

    @functools.partial(
        pl.kernel,
        out_type=(
            jax.ShapeDtypeStruct((N, D), jnp.float32),
            jax.ShapeDtypeStruct((N, D), jnp.float32),
        ),
        mesh=mesh,
        scratch_types=[
            pltpu.VMEM((4, CL), jnp.int32),
            pltpu.VMEM((CH, C), jnp.int32),
            pltpu.VMEM((2, CL, D), jnp.float32),
            pltpu.VMEM_SHARED((N, D), jnp.float32),
            pltpu.SemaphoreType.DMA,
            pltpu.SemaphoreType.DMA,
            pltpu.SemaphoreType.DMA,
            pltpu.SemaphoreType.DMA,
            pltpu.SemaphoreType.DMA,
            pltpu.SemaphoreType.DMA,
            pltpu.SemaphoreType.DMA,
            pltpu.SemaphoreType.DMA,
        ],
    )
    def sc_segment_sum(x_hbm, srci_hbm, dsti_hbm, zeros_hbm,
                       out0_hbm, out1_hbm,
                       iring, dsti_v, rows_v, acc,
                       isem0, isem1, isem2, isem3, gsem0, gsem1,
                       ssem0, ssem1):
        c = lax.axis_index("c")
        s = lax.axis_index("s")
        wid = c * NS + s
        isems = (isem0, isem1, isem2, isem3)
        gsems = (gsem0, gsem1)
        ssems = (ssem0, ssem1)

        def idx_fetch(jj, slot):
            return pltpu.make_async_copy(
                srci_hbm.at[wid, jj], iring.at[slot], isems[slot])

        def gather(jj_slot, b):
            return pltpu.make_async_copy(
                x_hbm.at[iring.at[jj_slot]], rows_v.at[b], gsems[b])

        def scatter_issue(jj, b):
            pltpu.async_copy(rows_v.at[b].at[pl.ds(0, C)],
                             acc.at[dsti_v.at[jj]], ssems[b], add=True)

        def scatter_wait(jj, b):
            pltpu.make_async_copy(rows_v.at[b].at[pl.ds(0, C)],
                                  acc.at[dsti_v.at[jj]], ssems[b]).wait()

        idx_fetch(0, 0).start()
        idx_fetch(1, 1).start()
        pltpu.sync_copy(dsti_hbm.at[wid], dsti_v)
        idx_fetch(0, 0).wait()
        gather(0, 0).start()

        @pl.when(s < NS - 1)
        def _():
            pltpu.sync_copy(zeros_hbm, acc.at[pl.ds(s * RPT, RPT)])

        @pl.when(s == NS - 1)
        def _():
            pltpu.sync_copy(zeros_hbm.at[pl.ds(0, RPT_LAST)],
                            acc.at[pl.ds((NS - 1) * RPT, RPT_LAST)])

        plsc.subcore_barrier()

        @pl.loop(0, CH, step=4, unroll=1)
        def _(j):
            for q in range(4):
                jj = j + q
                b = q % 2
                gather(q, b).wait()

                @pl.when(jj + 2 < CH)
                def _():
                    idx_fetch(jj + 2, (q + 2) % 4).start()

                scatter_issue(jj, b)

                @pl.when(jj + 1 < CH)
                def _():
                    idx_fetch(jj + 1, (q + 1) % 4).wait()

                    @pl.when(jj >= 1)
                    def _():
                        scatter_wait(jj - 1, 1 - b)

                    gather((q + 1) % 4, 1 - b).start()

        scatter_wait(CH - 1, (CH - 1) % 2)
        plsc.subcore_barrier()

        out_hbms = (out0_hbm, out1_hbm)
        for ci in range(NC):

            @pl.when(c == ci)
            def _(ci=ci):

                @pl.when(s < NS - 1)
                def _():
                    rows = pl.ds(s * RPT, RPT)
                    pltpu.sync_copy(acc.at[rows], out_hbms[ci].at[rows])

                @pl.when(s == NS - 1)
                def _():
                    rows = pl.ds((NS - 1) * RPT, RPT_LAST)
                    pltpu.sync_copy(acc.at[rows], out_hbms[ci].at[rows])

    return sc_segment_sum


_sc_segment_sum_cache = []


def _sc_segment_sum(*args):
    if not _sc_segment_sum_cache:
        _sc_segment_sum_cache.append(_build_sc_segment_sum())
    return _sc_segment_sum_cache[0](*args)


def _mlp_body(x_ref, a0_ref, a1_ref, w1_ref, b1_ref, w2_ref, b2_ref,
              g_ref, bt_ref, mn_ref, vr_ref, o_ref):
    a = x_ref[...] + a0_ref[...] + a1_ref[...]
    t = lax.dot_general(a, w1_ref[...], (((1,), (0,)), ((), ())),
                        preferred_element_type=jnp.float32)
    t = jnp.maximum(t + b1_ref[...], 0.0)
    h = lax.dot_general(t, w2_ref[...], (((1,), (0,)), ((), ())),
                        preferred_element_type=jnp.float32)
    h = h + b2_ref[...]
    h = (h - mn_ref[...]) * lax.rsqrt(vr_ref[...] + 1e-5) * g_ref[...] + bt_ref[...]
    o_ref[...] = jnp.maximum(h, 0.0)


_vec_spec = pl.BlockSpec((1, D), lambda i: (0, 0))
_w_spec = pl.BlockSpec((D, D), lambda i: (0, 0))
_row_spec = pl.BlockSpec((BLK, D), lambda i: (i, 0))

_mlp = pl.pallas_call(
    _mlp_body,
    grid=(N // BLK,),
    in_specs=[_row_spec, _row_spec, _row_spec,
              _w_spec, _vec_spec, _w_spec, _vec_spec,
              _vec_spec, _vec_spec, _vec_spec, _vec_spec],
    out_specs=_row_spec,
    out_shape=jax.ShapeDtypeStruct((N, D), jnp.float32),
)


def _mlp_pool_body(x_ref, a0_ref, a1_ref, w1_ref, b1_ref, w2_ref, b2_ref,
                   g_ref, bt_ref, mn_ref, vr_ref, wc_ref, bc_ref, o_ref):
    i = pl.program_id(0)
    a = x_ref[...] + a0_ref[...] + a1_ref[...]
    t = lax.dot_general(a, w1_ref[...], (((1,), (0,)), ((), ())),
                        preferred_element_type=jnp.float32)
    t = jnp.maximum(t + b1_ref[...], 0.0)
    h = lax.dot_general(t, w2_ref[...], (((1,), (0,)), ((), ())),
                        preferred_element_type=jnp.float32)
    h = h + b2_ref[...]
    h = (h - mn_ref[...]) * lax.rsqrt(vr_ref[...] + 1e-5) * g_ref[...] + bt_ref[...]
    h = jnp.maximum(h, 0.0)
    part = jnp.sum(h, axis=0, keepdims=True)

    @pl.when(i == 0)
    def _():
        o_ref[...] = part

    @pl.when(i > 0)
    def _():
        o_ref[...] = o_ref[...] + part

    @pl.when(i == N // BLK - 1)
    def _():
        pooled = o_ref[...]
        o_ref[...] = lax.dot_general(
            pooled, wc_ref[...], (((1,), (0,)), ((), ())),
            preferred_element_type=jnp.float32) + bc_ref[...]


_mlp_pool = pl.pallas_call(
    _mlp_pool_body,
    grid=(N // BLK,),
    in_specs=[_row_spec, _row_spec, _row_spec,
              _w_spec, _vec_spec, _w_spec, _vec_spec,
              _vec_spec, _vec_spec, _vec_spec, _vec_spec,
              _w_spec, _vec_spec],
    out_specs=pl.BlockSpec((1, D), lambda i: (0, 0)),
    out_shape=jax.ShapeDtypeStruct((1, D), jnp.float32),
)


def kernel(x, edge_index, W1_0, b1_0, W2_0, b2_0, gamma_0, beta_0, mean_0,
           var_0, W1_1, b1_1, W2_1, b2_1, gamma_1, beta_1, mean_1, var_1,
           W1_2, b1_2, W2_2, b2_2, gamma_2, beta_2, mean_2, var_2, Wc, bc):
    srci = jnp.concatenate(
        [edge_index[0].reshape(NW, CH, C),
         jnp.zeros((NW, CH, CL - C), jnp.int32)], axis=2)
    dsti = edge_index[1].reshape(NW, CH, C)
    zeros = jnp.zeros((RPT, D), jnp.float32)
    h = x

    params = [
        (W1_0, b1_0, W2_0, b2_0, gamma_0, beta_0, mean_0, var_0),
        (W1_1, b1_1, W2_1, b2_1, gamma_1, beta_1, mean_1, var_1),
        (W1_2, b1_2, W2_2, b2_2, gamma_2, beta_2, mean_2, var_2),
    ]

    for li, (W1, b1, W2, b2, g, bt, mn, vr) in enumerate(params):
        a0, a1 = _sc_segment_sum(h, srci, dsti, zeros)
        vecs = [v.reshape(1, D) for v in (b1, b2, g, bt, mn, vr)]
        if li < 2:
            h = _mlp(h, a0, a1, W1, vecs[0], W2, vecs[1],
                     vecs[2], vecs[3], vecs[4], vecs[5])
        else:
            wc_pad = jnp.concatenate(
                [Wc, jnp.zeros((D, D - D_OUT), jnp.float32)], axis=1)
            bc_pad = jnp.concatenate(
                [bc, jnp.zeros((D - D_OUT,), jnp.float32)]).reshape(1, D)
            out = _mlp_pool(h, a0, a1, W1, vecs[0], W2, vecs[1],
                            vecs[2], vecs[3], vecs[4], vecs[5],
                            wc_pad, bc_pad)
    return out[:, :D_OUT]
